# Initial kernel scaffold; baseline (speedup 1.0000x reference)
import jax
import jax.numpy as jnp
from jax.experimental import pallas as pl


def kernel(edge, emb_mat, W_scale, b_scale, W_att, b_att):
    raise NotImplementedError("write your pallas kernel here")


# speedup vs baseline: 13.7011x; 13.7011x over previous
import functools

import jax
import jax.numpy as jnp
from jax import lax
from jax.experimental import pallas as pl
from jax.experimental.pallas import tpu as pltpu
from jax.experimental.pallas import tpu_sc as plsc

N_RAW = 10001
D = 128
E = 320000
N_PAD = 10240
NC, NS, L = 2, 16, 16
NW = NC * NS
EPW = E // NW
CH = 80
NCHUNK = EPW // CH
RPT = N_PAD // NS
RPW = N_PAD // NW
RB = 1024

_MESH = plsc.VectorSubcoreMesh(
    core_axis_name="c", subcore_axis_name="s", num_cores=NC, num_subcores=NS)


def _dense_body(emb_ref, ws_ref, bs_ref, wab_ref, bab_ref, item_ref, aux_ref):
    y = jnp.dot(emb_ref[...], ws_ref[...],
                preferred_element_type=jnp.float32) + bs_ref[...]
    item_ref[...] = y
    aux_ref[...] = jnp.dot(y, wab_ref[...],
                           preferred_element_type=jnp.float32) + bab_ref[...]


def _dense(emb_p, W_scale, b_scale, Wab, bab):
    return pl.pallas_call(
        _dense_body,
        grid=(N_PAD // RB,),
        in_specs=[
            pl.BlockSpec((RB, D), lambda i: (i, 0)),
            pl.BlockSpec((D, D), lambda i: (0, 0)),
            pl.BlockSpec((1, D), lambda i: (0, 0)),
            pl.BlockSpec((D, D), lambda i: (0, 0)),
            pl.BlockSpec((1, D), lambda i: (0, 0)),
        ],
        out_specs=[
            pl.BlockSpec((RB, D), lambda i: (i, 0)),
            pl.BlockSpec((RB, D), lambda i: (i, 0)),
        ],
        out_shape=[
            jax.ShapeDtypeStruct((N_PAD, D), jnp.float32),
            jax.ShapeDtypeStruct((N_PAD, D), jnp.float32),
        ],
    )(emb_p, W_scale, b_scale, Wab, bab)


def _edge_body(src_hbm, dst_hbm, av_hbm, bv_hbm, item_hbm,
               num_out, den_out,
               av_v, bv_v, src_c, dst_c, s_v, rows_v,
               acc_sh, den_sh, sem):
    c = lax.axis_index("c")
    s = lax.axis_index("s")
    w = c * NS + s
    zf = jnp.zeros((L,), jnp.float32)

    def zrow(j, _):
        for k in range(D // L):
            rows_v[j, pl.ds(k * L, L)] = zf
        return 0
    lax.fori_loop(0, CH, zrow, 0, unroll=False)
    for k in range(CH // L):
        s_v[pl.ds(k * L, L)] = zf

    for k in range(RPT // CH):
        pltpu.sync_copy(rows_v, acc_sh.at[pl.ds(s * RPT + k * CH, CH)])
        pltpu.sync_copy(s_v, den_sh.at[pl.ds(s * RPT + k * CH, CH)])

    pltpu.sync_copy(av_hbm, av_v)
    pltpu.sync_copy(bv_hbm, bv_v)
    base = w * EPW

    plsc.subcore_barrier()

    def chunk(i, _):
        off = base + i * CH
        pltpu.sync_copy(src_hbm.at[pl.ds(off, CH)], src_c)
        pltpu.sync_copy(dst_hbm.at[pl.ds(off, CH)], dst_c)
        cp = pltpu.async_copy(item_hbm.at[dst_c], rows_v, sem)
        for k in range(CH // L):
            si = src_c[pl.ds(k * L, L)]
            di = dst_c[pl.ds(k * L, L)]
            att = plsc.load_gather(av_v, [si]) + plsc.load_gather(bv_v, [di])
            sc = jnp.exp(jnp.maximum(att, 0.2 * att) - 1.0)
            s_v[pl.ds(k * L, L)] = sc
        pltpu.sync_copy(s_v, den_sh.at[src_c], add=True)
        cp.wait()
        for j in range(CH):
            spl = plsc.load_gather(s_v, [jnp.full((L,), j, jnp.int32)])
            for k in range(D // L):
                rows_v[j, pl.ds(k * L, L)] = rows_v[j, pl.ds(k * L, L)] * spl
        pltpu.sync_copy(rows_v, acc_sh.at[src_c], add=True)
        return 0
    lax.fori_loop(0, NCHUNK, chunk, 0, unroll=False)

    plsc.subcore_barrier()

    pltpu.sync_copy(acc_sh.at[pl.ds(s * RPT, RPT)],
                    num_out.at[c, pl.ds(s * RPT, RPT)])
    pltpu.sync_copy(den_sh.at[pl.ds(s * RPT, RPT)],
                    den_out.at[pl.ds(c * N_PAD + s * RPT, RPT)])


_edge_kernel = functools.partial(
    pl.kernel,
    compiler_params=pltpu.CompilerParams(needs_layout_passes=False),
    out_type=(
        jax.ShapeDtypeStruct((NC, N_PAD, D), jnp.float32),
        jax.ShapeDtypeStruct((NC * N_PAD,), jnp.float32),
    ),
    mesh=_MESH,
    scratch_types=[
        pltpu.VMEM((N_PAD,), jnp.float32),
        pltpu.VMEM((N_PAD,), jnp.float32),
        pltpu.VMEM((CH,), jnp.int32),
        pltpu.VMEM((CH,), jnp.int32),
        pltpu.VMEM((CH,), jnp.float32),
        pltpu.VMEM((CH, D), jnp.float32),
        pltpu.VMEM_SHARED((N_PAD, D), jnp.float32),
        pltpu.VMEM_SHARED((N_PAD,), jnp.float32),
        pltpu.SemaphoreType.DMA,
    ],
)(_edge_body)


def _combine_body(numf_hbm, den_hbm, out_hbm, n0, n1, d0, d1, rd):
    c = lax.axis_index("c")
    s = lax.axis_index("s")
    w = c * NS + s
    base = w * RPW

    pltpu.sync_copy(numf_hbm.at[pl.ds(base * D, RPW * D)], n0)
    pltpu.sync_copy(numf_hbm.at[pl.ds(N_PAD * D + base * D, RPW * D)], n1)
    pltpu.sync_copy(den_hbm.at[pl.ds(base, RPW)], d0)
    pltpu.sync_copy(den_hbm.at[pl.ds(N_PAD + base, RPW)], d1)

    def dcalc(i, _):
        dsum = d0[pl.ds(i * L, L)] + d1[pl.ds(i * L, L)]
        rd[pl.ds(i * L, L)] = 1.0 / jnp.maximum(dsum, 1e-37)
        return 0
    lax.fori_loop(0, RPW // L, dcalc, 0, unroll=False)

    def row(j, _):
        spl = plsc.load_gather(rd, [jnp.zeros((L,), jnp.int32) + j])
        for k in range(D // L):
            o = j * D + k * L
            x = (n0[pl.ds(o, L)] + n1[pl.ds(o, L)]) * spl
            n0[pl.ds(o, L)] = 1.0 / (1.0 + jnp.exp(-x))
        return 0
    lax.fori_loop(0, RPW, row, 0, unroll=False)

    pltpu.sync_copy(n0, out_hbm.at[pl.ds(base * D, RPW * D)])


_combine_kernel = functools.partial(
    pl.kernel,
    compiler_params=pltpu.CompilerParams(needs_layout_passes=False),
    out_type=jax.ShapeDtypeStruct((N_PAD * D,), jnp.float32),
    mesh=_MESH,
    scratch_types=[
        pltpu.VMEM((RPW * D,), jnp.float32),
        pltpu.VMEM((RPW * D,), jnp.float32),
        pltpu.VMEM((RPW,), jnp.float32),
        pltpu.VMEM((RPW,), jnp.float32),
        pltpu.VMEM((RPW,), jnp.float32),
    ],
)(_combine_body)


def kernel(edge, emb_mat, W_scale, b_scale, W_att, b_att):
    src = edge[:, 0].astype(jnp.int32)
    dst = edge[:, 1].astype(jnp.int32)

    emb_p = jnp.pad(emb_mat, ((0, N_PAD - N_RAW), (0, 0)))
    Wab = jnp.zeros((D, D), jnp.float32)
    Wab = Wab.at[:, 0].set(W_att[:D, 0]).at[:, 1].set(W_att[D:, 0])
    bab = jnp.zeros((1, D), jnp.float32).at[0, 0].set(b_att[0])

    item, aux = _dense(emb_p, W_scale, b_scale.reshape(1, D), Wab, bab)
    av = aux[:, 0]
    bv = aux[:, 1]

    num, den = _edge_kernel(src, dst, av, bv, item)
    out = _combine_kernel(num.reshape(NC * N_PAD * D), den)
    return out.reshape(N_PAD, D)[:N_RAW]


# speedup vs baseline: 16.5336x; 1.2067x over previous
import functools

import jax
import jax.numpy as jnp
from jax import lax
from jax.experimental import pallas as pl
from jax.experimental.pallas import tpu as pltpu
from jax.experimental.pallas import tpu_sc as plsc

N_RAW = 10001
D = 128
E = 320000
N_PAD = 10240
NC, NS, L = 2, 16, 16
NW = NC * NS
EPW = E // NW
CH = 96
NCHUNK = -(-EPW // CH)
NPAIR = NCHUNK // 2
G = CH // L
E_PAD = NW * EPW + 128
RPT = N_PAD // NS
RPW = N_PAD // NW
RB = 1024

_MESH = plsc.VectorSubcoreMesh(
    core_axis_name="c", subcore_axis_name="s", num_cores=NC, num_subcores=NS)


def _dense_body(emb_ref, ws_ref, bs_ref, wab_ref, bab_ref, item_ref, aux_ref):
    y = jnp.dot(emb_ref[...], ws_ref[...],
                preferred_element_type=jnp.float32) + bs_ref[...]
    item_ref[...] = y
    aux_ref[...] = jnp.dot(y, wab_ref[...],
                           preferred_element_type=jnp.float32) + bab_ref[...]


def _dense(emb_p, W_scale, b_scale, Wab, bab):
    return pl.pallas_call(
        _dense_body,
        grid=(N_PAD // RB,),
        in_specs=[
            pl.BlockSpec((RB, D), lambda i: (i, 0)),
            pl.BlockSpec((D, D), lambda i: (0, 0)),
            pl.BlockSpec((1, D), lambda i: (0, 0)),
            pl.BlockSpec((D, D), lambda i: (0, 0)),
            pl.BlockSpec((1, D), lambda i: (0, 0)),
        ],
        out_specs=[
            pl.BlockSpec((RB, D), lambda i: (i, 0)),
            pl.BlockSpec((RB, D), lambda i: (i, 0)),
        ],
        out_shape=[
            jax.ShapeDtypeStruct((N_PAD, D), jnp.float32),
            jax.ShapeDtypeStruct((N_PAD, D), jnp.float32),
        ],
    )(emb_p, W_scale, b_scale, Wab, bab)


def _sigmoid_body(x_ref, o_ref):
    o_ref[...] = jax.nn.sigmoid(x_ref[...])


def _sigmoid(x):
    return pl.pallas_call(
        _sigmoid_body,
        grid=(N_PAD // RB,),
        in_specs=[pl.BlockSpec((RB, D), lambda i: (i, 0))],
        out_specs=pl.BlockSpec((RB, D), lambda i: (i, 0)),
        out_shape=jax.ShapeDtypeStruct((N_PAD, D), jnp.float32),
    )(x)


def _edge_body(src_hbm, dst_hbm, av_hbm, bv_hbm, item_hbm,
               num_out, den_out,
               av_v, bv_v, src_c, dst_c, scat_c, s_v, rows_v,
               acc_sh, den_sh, sem_ix, sem_g, sem_d, sem_s):
    c = lax.axis_index("c")
    s = lax.axis_index("s")
    w = c * NS + s
    base = w * EPW
    zf = jnp.zeros((L,), jnp.float32)

    def idx_start(i, p):
        off = base + i * CH
        pltpu.async_copy(src_hbm.at[pl.ds(off, CH)], src_c[p], sem_ix[p])
        pltpu.async_copy(dst_hbm.at[pl.ds(off, CH)], dst_c[p], sem_ix[p])

    def idx_wait(p):
        pltpu.make_async_copy(src_hbm.at[pl.ds(0, CH)], src_c[p],
                              sem_ix[p]).wait()
        pltpu.make_async_copy(dst_hbm.at[pl.ds(0, CH)], dst_c[p],
                              sem_ix[p]).wait()

    def gather_start(p):
        pltpu.async_copy(item_hbm.at[dst_c[p]], rows_v[p], sem_g[p])

    def gather_wait(p):
        pltpu.make_async_copy(item_hbm.at[dst_c[p]], rows_v[p],
                              sem_g[p]).wait()

    def den_start(p):
        pltpu.async_copy(s_v[p], den_sh.at[scat_c[p]], sem_d[p], add=True)

    def den_wait(p):
        pltpu.make_async_copy(s_v[p], den_sh.at[scat_c[p]], sem_d[p]).wait()

    def scat_start(p):
        pltpu.async_copy(rows_v[p], acc_sh.at[scat_c[p]], sem_s[p], add=True)

    def scat_wait(p):
        pltpu.make_async_copy(rows_v[p], acc_sh.at[scat_c[p]],
                              sem_s[p]).wait()

    def compute_s(i, p):
        limit = EPW - i * CH
        for k in range(G):
            si = src_c[p][pl.ds(k * L, L)]
            di = dst_c[p][pl.ds(k * L, L)]
            scat_c[p][pl.ds(k * L, L)] = si
            att = plsc.load_gather(av_v, [si]) + plsc.load_gather(bv_v, [di])
            sc = jnp.exp(jnp.maximum(att, 0.2 * att) - 1.0)
            mask = (lax.iota(jnp.int32, L) + k * L) < limit
            s_v[p][pl.ds(k * L, L)] = jnp.where(mask, sc, 0.0)

    def scale_rows(p):
        for j in range(CH):
            spl = plsc.load_gather(s_v[p], [jnp.full((L,), j, jnp.int32)])
            for k in range(D // L):
                rows_v[p][j, pl.ds(k * L, L)] = (
                    rows_v[p][j, pl.ds(k * L, L)] * spl)

    def zrow(j, _):
        for k in range(D // L):
            rows_v[0][j, pl.ds(k * L, L)] = zf
        return 0
    lax.fori_loop(0, CH, zrow, 0, unroll=False)
    for k in range(G):
        s_v[0][pl.ds(k * L, L)] = zf
    for k in range(6):
        pltpu.sync_copy(rows_v[0], acc_sh.at[pl.ds(s * RPT + k * CH, CH)])
        pltpu.sync_copy(s_v[0], den_sh.at[pl.ds(s * RPT + k * CH, CH)])
    pltpu.sync_copy(rows_v[0].at[pl.ds(0, RPT - 6 * CH)],
                    acc_sh.at[pl.ds(s * RPT + 6 * CH, RPT - 6 * CH)])
    pltpu.sync_copy(s_v[0].at[pl.ds(0, RPT - 6 * CH)],
                    den_sh.at[pl.ds(s * RPT + 6 * CH, RPT - 6 * CH)])

    pltpu.sync_copy(av_hbm, av_v)
    pltpu.sync_copy(bv_hbm, bv_v)
    idx_start(0, 0)
    idx_start(1, 1)
    idx_wait(0)
    gather_start(0)

    plsc.subcore_barrier()

    def pair(t, _):
        i = 2 * t
        idx_wait(1)

        @pl.when(t >= 1)
        def _():
            scat_wait(1)
        gather_start(1)

        @pl.when(t >= 1)
        def _():
            den_wait(0)
        compute_s(i, 0)
        den_start(0)
        gather_wait(0)
        scale_rows(0)
        scat_start(0)
        idx_start(i + 2, 0)

        idx_wait(0)
        scat_wait(0)
        gather_start(0)

        @pl.when(t >= 1)
        def _():
            den_wait(1)
        compute_s(i + 1, 1)
        den_start(1)
        gather_wait(1)
        scale_rows(1)
        scat_start(1)

        @pl.when(t <= NPAIR - 2)
        def _():
            idx_start(i + 3, 1)
        return 0
    lax.fori_loop(0, NPAIR, pair, 0, unroll=False)

    scat_wait(1)
    den_wait(0)
    compute_s(NCHUNK - 1, 0)
    den_start(0)
    gather_wait(0)
    scale_rows(0)
    scat_start(0)
    den_wait(1)
    den_wait(0)
    scat_wait(0)

    plsc.subcore_barrier()

    pltpu.sync_copy(acc_sh.at[pl.ds(s * RPT, RPT)],
                    num_out.at[c, pl.ds(s * RPT, RPT)])
    pltpu.sync_copy(den_sh.at[pl.ds(s * RPT, RPT)],
                    den_out.at[pl.ds(c * N_PAD + s * RPT, RPT)])


_edge_kernel = functools.partial(
    pl.kernel,
    compiler_params=pltpu.CompilerParams(needs_layout_passes=False),
    out_type=(
        jax.ShapeDtypeStruct((NC, N_PAD, D), jnp.float32),
        jax.ShapeDtypeStruct((NC * N_PAD,), jnp.float32),
    ),
    mesh=_MESH,
    scratch_types=[
        pltpu.VMEM((N_PAD,), jnp.float32),
        pltpu.VMEM((N_PAD,), jnp.float32),
        [pltpu.VMEM((CH,), jnp.int32)] * 2,
        [pltpu.VMEM((CH,), jnp.int32)] * 2,
        [pltpu.VMEM((CH,), jnp.int32)] * 2,
        [pltpu.VMEM((CH,), jnp.float32)] * 2,
        [pltpu.VMEM((CH, D), jnp.float32)] * 2,
        pltpu.VMEM_SHARED((N_PAD, D), jnp.float32),
        pltpu.VMEM_SHARED((N_PAD,), jnp.float32),
        [pltpu.SemaphoreType.DMA] * 2,
        [pltpu.SemaphoreType.DMA] * 2,
        [pltpu.SemaphoreType.DMA] * 2,
        [pltpu.SemaphoreType.DMA] * 2,
    ],
)(_edge_body)


def _combine_body(numf_hbm, den_hbm, out_hbm, n0, n1, d0, d1, rd):
    c = lax.axis_index("c")
    s = lax.axis_index("s")
    w = c * NS + s
    base = w * RPW

    pltpu.sync_copy(numf_hbm.at[pl.ds(base * D, RPW * D)], n0)
    pltpu.sync_copy(numf_hbm.at[pl.ds(N_PAD * D + base * D, RPW * D)], n1)
    pltpu.sync_copy(den_hbm.at[pl.ds(base, RPW)], d0)
    pltpu.sync_copy(den_hbm.at[pl.ds(N_PAD + base, RPW)], d1)

    def dcalc(i, _):
        dsum = d0[pl.ds(i * L, L)] + d1[pl.ds(i * L, L)]
        rd[pl.ds(i * L, L)] = 1.0 / jnp.maximum(dsum, 1e-37)
        return 0
    lax.fori_loop(0, RPW // L, dcalc, 0, unroll=False)

    def row(j, _):
        spl = plsc.load_gather(rd, [jnp.zeros((L,), jnp.int32) + j])
        for k in range(D // L):
            o = j * D + k * L
            n0[pl.ds(o, L)] = (n0[pl.ds(o, L)] + n1[pl.ds(o, L)]) * spl
        return 0
    lax.fori_loop(0, RPW, row, 0, unroll=False)

    pltpu.sync_copy(n0, out_hbm.at[pl.ds(base * D, RPW * D)])


_combine_kernel = functools.partial(
    pl.kernel,
    compiler_params=pltpu.CompilerParams(needs_layout_passes=False),
    out_type=jax.ShapeDtypeStruct((N_PAD * D,), jnp.float32),
    mesh=_MESH,
    scratch_types=[
        pltpu.VMEM((RPW * D,), jnp.float32),
        pltpu.VMEM((RPW * D,), jnp.float32),
        pltpu.VMEM((RPW,), jnp.float32),
        pltpu.VMEM((RPW,), jnp.float32),
        pltpu.VMEM((RPW,), jnp.float32),
    ],
)(_combine_body)


def kernel(edge, emb_mat, W_scale, b_scale, W_att, b_att):
    src = edge[:, 0].astype(jnp.int32)
    dst = edge[:, 1].astype(jnp.int32)
    src = jnp.pad(src, (0, E_PAD - E))
    dst = jnp.pad(dst, (0, E_PAD - E))

    emb_p = jnp.pad(emb_mat, ((0, N_PAD - N_RAW), (0, 0)))
    Wab = jnp.zeros((D, D), jnp.float32)
    Wab = Wab.at[:, 0].set(W_att[:D, 0]).at[:, 1].set(W_att[D:, 0])
    bab = jnp.zeros((1, D), jnp.float32).at[0, 0].set(b_att[0])

    item, aux = _dense(emb_p, W_scale, b_scale.reshape(1, D), Wab, bab)
    av = aux[:, 0]
    bv = aux[:, 1]

    num, den = _edge_kernel(src, dst, av, bv, item)
    x = _combine_kernel(num.reshape(NC * N_PAD * D), den)
    out = _sigmoid(x.reshape(N_PAD, D))
    return out[:N_RAW]


# speedup vs baseline: 23.2527x; 1.4064x over previous
import functools

import jax
import jax.numpy as jnp
from jax import lax
from jax.experimental import pallas as pl
from jax.experimental.pallas import tpu as pltpu
from jax.experimental.pallas import tpu_sc as plsc

N_RAW = 10001
D = 128
E = 320000
N_PAD = 10240
NC, NS, L = 2, 16, 16
NW = NC * NS
EPW = E // NW
CH = 96
NCHUNK = -(-EPW // CH)
NPAIR = NCHUNK // 2
G = CH // L
E_PAD = NW * EPW + 128
RPT = N_PAD // NS
RPW = N_PAD // NW
RB = 1024

_MESH = plsc.VectorSubcoreMesh(
    core_axis_name="c", subcore_axis_name="s", num_cores=NC, num_subcores=NS)


def _dense_body(emb_ref, ws_ref, bs_ref, wab_ref, bab_ref, item_ref, aux_ref):
    y = jnp.dot(emb_ref[...], ws_ref[...],
                preferred_element_type=jnp.float32) + bs_ref[...]
    item_ref[...] = y
    aux_ref[...] = jnp.dot(y, wab_ref[...],
                           preferred_element_type=jnp.float32) + bab_ref[...]


def _dense(emb_p, W_scale, b_scale, Wab, bab):
    return pl.pallas_call(
        _dense_body,
        grid=(N_PAD // RB,),
        in_specs=[
            pl.BlockSpec((RB, D), lambda i: (i, 0)),
            pl.BlockSpec((D, D), lambda i: (0, 0)),
            pl.BlockSpec((1, D), lambda i: (0, 0)),
            pl.BlockSpec((D, D), lambda i: (0, 0)),
            pl.BlockSpec((1, D), lambda i: (0, 0)),
        ],
        out_specs=[
            pl.BlockSpec((RB, D), lambda i: (i, 0)),
            pl.BlockSpec((RB, D), lambda i: (i, 0)),
        ],
        out_shape=[
            jax.ShapeDtypeStruct((N_PAD, D), jnp.float32),
            jax.ShapeDtypeStruct((N_PAD, D), jnp.float32),
        ],
    )(emb_p, W_scale, b_scale, Wab, bab)


def _sigmoid_body(x_ref, o_ref):
    o_ref[...] = jax.nn.sigmoid(x_ref[...])


def _sigmoid(x):
    return pl.pallas_call(
        _sigmoid_body,
        grid=(N_PAD // RB,),
        in_specs=[pl.BlockSpec((RB, D), lambda i: (i, 0))],
        out_specs=pl.BlockSpec((RB, D), lambda i: (i, 0)),
        out_shape=jax.ShapeDtypeStruct((N_PAD, D), jnp.float32),
    )(x)


def _edge_body(src_hbm, dst_hbm, av_hbm, bv_hbm, item_hbm,
               num_out, den_out,
               av_v, bv_v, src_c, dst_c, scat_c, s_v, rows_v,
               acc_sh, den_sh, sem_ix, sem_g, sem_d, sem_s):
    c = lax.axis_index("c")
    s = lax.axis_index("s")
    w = c * NS + s
    base = w * EPW
    zf = jnp.zeros((L,), jnp.float32)

    def idx_start(i, p):
        off = base + i * CH
        pltpu.async_copy(src_hbm.at[pl.ds(off, CH)], src_c[p], sem_ix[p])
        pltpu.async_copy(dst_hbm.at[pl.ds(off, CH)], dst_c[p], sem_ix[p])

    def idx_wait(p):
        pltpu.make_async_copy(src_hbm.at[pl.ds(0, CH)], src_c[p],
                              sem_ix[p]).wait()
        pltpu.make_async_copy(dst_hbm.at[pl.ds(0, CH)], dst_c[p],
                              sem_ix[p]).wait()

    def gather_start(p):
        pltpu.async_copy(item_hbm.at[dst_c[p]], rows_v[p], sem_g[p])

    def gather_wait(p):
        pltpu.make_async_copy(item_hbm.at[dst_c[p]], rows_v[p],
                              sem_g[p]).wait()

    def den_start(p):
        pltpu.async_copy(s_v[p], den_sh.at[scat_c[p]], sem_d[p], add=True)

    def den_wait(p):
        pltpu.make_async_copy(s_v[p], den_sh.at[scat_c[p]], sem_d[p]).wait()

    def scat_start(p):
        pltpu.async_copy(rows_v[p], acc_sh.at[scat_c[p]], sem_s[p], add=True)

    def scat_wait(p):
        pltpu.make_async_copy(rows_v[p], acc_sh.at[scat_c[p]],
                              sem_s[p]).wait()

    def compute_s(i, p):
        limit = EPW - i * CH
        for k in range(G):
            si = src_c[p][pl.ds(k * L, L)]
            di = dst_c[p][pl.ds(k * L, L)]
            scat_c[p][pl.ds(k * L, L)] = si
            att = plsc.load_gather(av_v, [si]) + plsc.load_gather(bv_v, [di])
            sc = jnp.exp(jnp.maximum(att, 0.2 * att) - 1.0)
            mask = (lax.iota(jnp.int32, L) + k * L) < limit
            s_v[p][pl.ds(k * L, L)] = jnp.where(mask, sc, 0.0)

    def scale_rows(p):
        for g in range(G):
            sv = s_v[p][pl.ds(g * L, L)]
            for jj in range(L):
                j = g * L + jj
                ssc = sv[jj]
                for k in range(D // L):
                    rows_v[p][j, pl.ds(k * L, L)] = (
                        rows_v[p][j, pl.ds(k * L, L)] * ssc)

    def zrow(j, _):
        for k in range(D // L):
            rows_v[0][j, pl.ds(k * L, L)] = zf
        return 0
    lax.fori_loop(0, CH, zrow, 0, unroll=False)
    for k in range(G):
        s_v[0][pl.ds(k * L, L)] = zf
    for k in range(6):
        pltpu.sync_copy(rows_v[0], acc_sh.at[pl.ds(s * RPT + k * CH, CH)])
        pltpu.sync_copy(s_v[0], den_sh.at[pl.ds(s * RPT + k * CH, CH)])
    pltpu.sync_copy(rows_v[0].at[pl.ds(0, RPT - 6 * CH)],
                    acc_sh.at[pl.ds(s * RPT + 6 * CH, RPT - 6 * CH)])
    pltpu.sync_copy(s_v[0].at[pl.ds(0, RPT - 6 * CH)],
                    den_sh.at[pl.ds(s * RPT + 6 * CH, RPT - 6 * CH)])

    pltpu.sync_copy(av_hbm, av_v)
    pltpu.sync_copy(bv_hbm, bv_v)
    idx_start(0, 0)
    idx_start(1, 1)
    idx_wait(0)
    gather_start(0)

    plsc.subcore_barrier()

    def pair(t, _):
        i = 2 * t
        idx_wait(1)

        @pl.when(t >= 1)
        def _():
            scat_wait(1)
        gather_start(1)

        @pl.when(t >= 1)
        def _():
            den_wait(0)
        compute_s(i, 0)
        den_start(0)
        gather_wait(0)
        scale_rows(0)
        scat_start(0)
        idx_start(i + 2, 0)

        idx_wait(0)
        scat_wait(0)
        gather_start(0)

        @pl.when(t >= 1)
        def _():
            den_wait(1)
        compute_s(i + 1, 1)
        den_start(1)
        gather_wait(1)
        scale_rows(1)
        scat_start(1)

        @pl.when(t <= NPAIR - 2)
        def _():
            idx_start(i + 3, 1)
        return 0
    lax.fori_loop(0, NPAIR, pair, 0, unroll=False)

    scat_wait(1)
    den_wait(0)
    compute_s(NCHUNK - 1, 0)
    den_start(0)
    gather_wait(0)
    scale_rows(0)
    scat_start(0)
    den_wait(1)
    den_wait(0)
    scat_wait(0)

    plsc.subcore_barrier()

    pltpu.sync_copy(acc_sh.at[pl.ds(s * RPT, RPT)],
                    num_out.at[c, pl.ds(s * RPT, RPT)])
    pltpu.sync_copy(den_sh.at[pl.ds(s * RPT, RPT)],
                    den_out.at[pl.ds(c * N_PAD + s * RPT, RPT)])


_edge_kernel = functools.partial(
    pl.kernel,
    compiler_params=pltpu.CompilerParams(needs_layout_passes=False),
    out_type=(
        jax.ShapeDtypeStruct((NC, N_PAD, D), jnp.float32),
        jax.ShapeDtypeStruct((NC * N_PAD,), jnp.float32),
    ),
    mesh=_MESH,
    scratch_types=[
        pltpu.VMEM((N_PAD,), jnp.float32),
        pltpu.VMEM((N_PAD,), jnp.float32),
        [pltpu.VMEM((CH,), jnp.int32)] * 2,
        [pltpu.VMEM((CH,), jnp.int32)] * 2,
        [pltpu.VMEM((CH,), jnp.int32)] * 2,
        [pltpu.VMEM((CH,), jnp.float32)] * 2,
        [pltpu.VMEM((CH, D), jnp.float32)] * 2,
        pltpu.VMEM_SHARED((N_PAD, D), jnp.float32),
        pltpu.VMEM_SHARED((N_PAD,), jnp.float32),
        [pltpu.SemaphoreType.DMA] * 2,
        [pltpu.SemaphoreType.DMA] * 2,
        [pltpu.SemaphoreType.DMA] * 2,
        [pltpu.SemaphoreType.DMA] * 2,
    ],
)(_edge_body)


def _combine_body(numf_hbm, den_hbm, out_hbm, n0, n1, d0, d1, rd):
    c = lax.axis_index("c")
    s = lax.axis_index("s")
    w = c * NS + s
    base = w * RPW

    pltpu.sync_copy(numf_hbm.at[pl.ds(base * D, RPW * D)], n0)
    pltpu.sync_copy(numf_hbm.at[pl.ds(N_PAD * D + base * D, RPW * D)], n1)
    pltpu.sync_copy(den_hbm.at[pl.ds(base, RPW)], d0)
    pltpu.sync_copy(den_hbm.at[pl.ds(N_PAD + base, RPW)], d1)

    def dcalc(i, _):
        dsum = d0[pl.ds(i * L, L)] + d1[pl.ds(i * L, L)]
        rd[pl.ds(i * L, L)] = 1.0 / jnp.maximum(dsum, 1e-37)
        return 0
    lax.fori_loop(0, RPW // L, dcalc, 0, unroll=False)

    def row(j, _):
        spl = plsc.load_gather(rd, [jnp.zeros((L,), jnp.int32) + j])
        for k in range(D // L):
            o = j * D + k * L
            n0[pl.ds(o, L)] = (n0[pl.ds(o, L)] + n1[pl.ds(o, L)]) * spl
        return 0
    lax.fori_loop(0, RPW, row, 0, unroll=False)

    pltpu.sync_copy(n0, out_hbm.at[pl.ds(base * D, RPW * D)])


_combine_kernel = functools.partial(
    pl.kernel,
    compiler_params=pltpu.CompilerParams(needs_layout_passes=False),
    out_type=jax.ShapeDtypeStruct((N_PAD * D,), jnp.float32),
    mesh=_MESH,
    scratch_types=[
        pltpu.VMEM((RPW * D,), jnp.float32),
        pltpu.VMEM((RPW * D,), jnp.float32),
        pltpu.VMEM((RPW,), jnp.float32),
        pltpu.VMEM((RPW,), jnp.float32),
        pltpu.VMEM((RPW,), jnp.float32),
    ],
)(_combine_body)


def kernel(edge, emb_mat, W_scale, b_scale, W_att, b_att):
    src = edge[:, 0].astype(jnp.int32)
    dst = edge[:, 1].astype(jnp.int32)
    src = jnp.pad(src, (0, E_PAD - E))
    dst = jnp.pad(dst, (0, E_PAD - E))

    emb_p = jnp.pad(emb_mat, ((0, N_PAD - N_RAW), (0, 0)))
    Wab = jnp.zeros((D, D), jnp.float32)
    Wab = Wab.at[:, 0].set(W_att[:D, 0]).at[:, 1].set(W_att[D:, 0])
    bab = jnp.zeros((1, D), jnp.float32).at[0, 0].set(b_att[0])

    item, aux = _dense(emb_p, W_scale, b_scale.reshape(1, D), Wab, bab)
    av = aux[:, 0]
    bv = aux[:, 1]

    num, den = _edge_kernel(src, dst, av, bv, item)
    x = _combine_kernel(num.reshape(NC * N_PAD * D), den)
    out = _sigmoid(x.reshape(N_PAD, D))
    return out[:N_RAW]


# speedup vs baseline: 24.0373x; 1.0337x over previous
import functools

import jax
import jax.numpy as jnp
from jax import lax
from jax.experimental import pallas as pl
from jax.experimental.pallas import tpu as pltpu
from jax.experimental.pallas import tpu_sc as plsc

N_RAW = 10001
D = 128
E = 320000
N_PAD = 10240
NC, NS, L = 2, 16, 16
NW = NC * NS
EPW = E // NW
CH = 96
NCHUNK = -(-EPW // CH)
NPAIR = NCHUNK // 2
G = CH // L
E_PAD = NW * EPW + 128
RPT = N_PAD // NS
RPW = N_PAD // NW
RB = 1024

_MESH = plsc.VectorSubcoreMesh(
    core_axis_name="c", subcore_axis_name="s", num_cores=NC, num_subcores=NS)


def _dense_body(emb_ref, ws_ref, bs_ref, wab_ref, bab_ref, item_ref, aux_ref):
    y = jnp.dot(emb_ref[...], ws_ref[...],
                preferred_element_type=jnp.float32) + bs_ref[...]
    item_ref[...] = y
    aux_ref[...] = jnp.dot(y, wab_ref[...],
                           preferred_element_type=jnp.float32) + bab_ref[...]


def _dense(emb_p, W_scale, b_scale, Wab, bab):
    return pl.pallas_call(
        _dense_body,
        grid=(N_PAD // RB,),
        in_specs=[
            pl.BlockSpec((RB, D), lambda i: (i, 0)),
            pl.BlockSpec((D, D), lambda i: (0, 0)),
            pl.BlockSpec((1, D), lambda i: (0, 0)),
            pl.BlockSpec((D, D), lambda i: (0, 0)),
            pl.BlockSpec((1, D), lambda i: (0, 0)),
        ],
        out_specs=[
            pl.BlockSpec((RB, D), lambda i: (i, 0)),
            pl.BlockSpec((RB, D), lambda i: (i, 0)),
        ],
        out_shape=[
            jax.ShapeDtypeStruct((N_PAD, D), jnp.float32),
            jax.ShapeDtypeStruct((N_PAD, D), jnp.float32),
        ],
    )(emb_p, W_scale, b_scale, Wab, bab)


def _finish_body(num_ref, den_ref, o_ref):
    n = num_ref[0] + num_ref[1]
    d = den_ref[0] + den_ref[1]
    rd = 1.0 / jnp.maximum(d, 1e-37)
    o_ref[...] = jax.nn.sigmoid(n * rd)


def _finish(num, den2):
    return pl.pallas_call(
        _finish_body,
        grid=(N_PAD // RB,),
        in_specs=[
            pl.BlockSpec((NC, RB, D), lambda i: (0, i, 0)),
            pl.BlockSpec((NC, RB, 1), lambda i: (0, i, 0)),
        ],
        out_specs=pl.BlockSpec((RB, D), lambda i: (i, 0)),
        out_shape=jax.ShapeDtypeStruct((N_PAD, D), jnp.float32),
    )(num, den2)


def _edge_body(src_hbm, dst_hbm, av_hbm, bv_hbm, item_hbm,
               num_out, den_out,
               av_v, bv_v, src_c, dst_c, scat_c, s_v, rows_v,
               acc_sh, den_sh, sem_ix, sem_g, sem_d, sem_s):
    c = lax.axis_index("c")
    s = lax.axis_index("s")
    w = c * NS + s
    base = w * EPW
    zf = jnp.zeros((L,), jnp.float32)

    def idx_start(i, p):
        off = base + i * CH
        pltpu.async_copy(src_hbm.at[pl.ds(off, CH)], src_c[p], sem_ix[p])
        pltpu.async_copy(dst_hbm.at[pl.ds(off, CH)], dst_c[p], sem_ix[p])

    def idx_wait(p):
        pltpu.make_async_copy(src_hbm.at[pl.ds(0, CH)], src_c[p],
                              sem_ix[p]).wait()
        pltpu.make_async_copy(dst_hbm.at[pl.ds(0, CH)], dst_c[p],
                              sem_ix[p]).wait()

    def gather_start(p):
        pltpu.async_copy(item_hbm.at[dst_c[p]], rows_v[p], sem_g[p])

    def gather_wait(p):
        pltpu.make_async_copy(item_hbm.at[dst_c[p]], rows_v[p],
                              sem_g[p]).wait()

    def den_start(p):
        pltpu.async_copy(s_v[p], den_sh.at[scat_c[p]], sem_d[p], add=True)

    def den_wait(p):
        pltpu.make_async_copy(s_v[p], den_sh.at[scat_c[p]], sem_d[p]).wait()

    def scat_start(p):
        pltpu.async_copy(rows_v[p], acc_sh.at[scat_c[p]], sem_s[p], add=True)

    def scat_wait(p):
        pltpu.make_async_copy(rows_v[p], acc_sh.at[scat_c[p]],
                              sem_s[p]).wait()

    def compute_s(i, p):
        limit = EPW - i * CH
        for k in range(G):
            si = src_c[p][pl.ds(k * L, L)]
            di = dst_c[p][pl.ds(k * L, L)]
            scat_c[p][pl.ds(k * L, L)] = si
            att = plsc.load_gather(av_v, [si]) + plsc.load_gather(bv_v, [di])
            sc = jnp.exp(jnp.maximum(att, 0.2 * att) - 1.0)
            mask = (lax.iota(jnp.int32, L) + k * L) < limit
            s_v[p][pl.ds(k * L, L)] = jnp.where(mask, sc, 0.0)

    def scale_rows(p):
        for g in range(G):
            sv = s_v[p][pl.ds(g * L, L)]
            for jj in range(L):
                j = g * L + jj
                ssc = sv[jj]
                for k in range(D // L):
                    rows_v[p][j, pl.ds(k * L, L)] = (
                        rows_v[p][j, pl.ds(k * L, L)] * ssc)

    def zrow(j, _):
        for k in range(D // L):
            rows_v[0][j, pl.ds(k * L, L)] = zf
        return 0
    lax.fori_loop(0, CH, zrow, 0, unroll=False)
    for k in range(G):
        s_v[0][pl.ds(k * L, L)] = zf
    for k in range(6):
        pltpu.sync_copy(rows_v[0], acc_sh.at[pl.ds(s * RPT + k * CH, CH)])
        pltpu.sync_copy(s_v[0], den_sh.at[pl.ds(s * RPT + k * CH, CH)])
    pltpu.sync_copy(rows_v[0].at[pl.ds(0, RPT - 6 * CH)],
                    acc_sh.at[pl.ds(s * RPT + 6 * CH, RPT - 6 * CH)])
    pltpu.sync_copy(s_v[0].at[pl.ds(0, RPT - 6 * CH)],
                    den_sh.at[pl.ds(s * RPT + 6 * CH, RPT - 6 * CH)])

    pltpu.sync_copy(av_hbm, av_v)
    pltpu.sync_copy(bv_hbm, bv_v)
    idx_start(0, 0)
    idx_start(1, 1)
    idx_wait(0)
    gather_start(0)

    plsc.subcore_barrier()

    def pair(t, _):
        i = 2 * t
        idx_wait(1)

        @pl.when(t >= 1)
        def _():
            scat_wait(1)
        gather_start(1)

        @pl.when(t >= 1)
        def _():
            den_wait(0)
        compute_s(i, 0)
        den_start(0)
        gather_wait(0)
        scale_rows(0)
        scat_start(0)
        idx_start(i + 2, 0)

        idx_wait(0)
        scat_wait(0)
        gather_start(0)

        @pl.when(t >= 1)
        def _():
            den_wait(1)
        compute_s(i + 1, 1)
        den_start(1)
        gather_wait(1)
        scale_rows(1)
        scat_start(1)

        @pl.when(t <= NPAIR - 2)
        def _():
            idx_start(i + 3, 1)
        return 0
    lax.fori_loop(0, NPAIR, pair, 0, unroll=False)

    scat_wait(1)
    den_wait(0)
    compute_s(NCHUNK - 1, 0)
    den_start(0)
    gather_wait(0)
    scale_rows(0)
    scat_start(0)
    den_wait(1)
    den_wait(0)
    scat_wait(0)

    plsc.subcore_barrier()

    pltpu.sync_copy(acc_sh.at[pl.ds(s * RPT, RPT)],
                    num_out.at[c, pl.ds(s * RPT, RPT)])
    pltpu.sync_copy(den_sh.at[pl.ds(s * RPT, RPT)],
                    den_out.at[pl.ds(c * N_PAD + s * RPT, RPT)])


_edge_kernel = functools.partial(
    pl.kernel,
    compiler_params=pltpu.CompilerParams(needs_layout_passes=False),
    out_type=(
        jax.ShapeDtypeStruct((NC, N_PAD, D), jnp.float32),
        jax.ShapeDtypeStruct((NC * N_PAD,), jnp.float32),
    ),
    mesh=_MESH,
    scratch_types=[
        pltpu.VMEM((N_PAD,), jnp.float32),
        pltpu.VMEM((N_PAD,), jnp.float32),
        [pltpu.VMEM((CH,), jnp.int32)] * 2,
        [pltpu.VMEM((CH,), jnp.int32)] * 2,
        [pltpu.VMEM((CH,), jnp.int32)] * 2,
        [pltpu.VMEM((CH,), jnp.float32)] * 2,
        [pltpu.VMEM((CH, D), jnp.float32)] * 2,
        pltpu.VMEM_SHARED((N_PAD, D), jnp.float32),
        pltpu.VMEM_SHARED((N_PAD,), jnp.float32),
        [pltpu.SemaphoreType.DMA] * 2,
        [pltpu.SemaphoreType.DMA] * 2,
        [pltpu.SemaphoreType.DMA] * 2,
        [pltpu.SemaphoreType.DMA] * 2,
    ],
)(_edge_body)


def kernel(edge, emb_mat, W_scale, b_scale, W_att, b_att):
    src = edge[:, 0].astype(jnp.int32)
    dst = edge[:, 1].astype(jnp.int32)
    src = jnp.pad(src, (0, E_PAD - E))
    dst = jnp.pad(dst, (0, E_PAD - E))

    emb_p = jnp.pad(emb_mat, ((0, N_PAD - N_RAW), (0, 0)))
    Wab = jnp.zeros((D, D), jnp.float32)
    Wab = Wab.at[:, 0].set(W_att[:D, 0]).at[:, 1].set(W_att[D:, 0])
    bab = jnp.zeros((1, D), jnp.float32).at[0, 0].set(b_att[0])

    item, aux = _dense(emb_p, W_scale, b_scale.reshape(1, D), Wab, bab)
    av = aux[:, 0]
    bv = aux[:, 1]

    num, den = _edge_kernel(src, dst, av, bv, item)
    out = _finish(num, den.reshape(NC, N_PAD, 1))
    return out[:N_RAW]


# speedup vs baseline: 24.3942x; 1.0148x over previous
import functools

import jax
import jax.numpy as jnp
from jax import lax
from jax.experimental import pallas as pl
from jax.experimental.pallas import tpu as pltpu
from jax.experimental.pallas import tpu_sc as plsc

N_RAW = 10001
D = 128
E = 320000
N_PAD = 10240
NC, NS, L = 2, 16, 16
NW = NC * NS
EPW = E // NW
CH = 96
NCHUNK = -(-EPW // CH)
NPAIR = NCHUNK // 2
G = CH // L
E_PAD = NW * EPW + 128
RPT = N_PAD // NS
RPW = N_PAD // NW
RB = 1024

_MESH = plsc.VectorSubcoreMesh(
    core_axis_name="c", subcore_axis_name="s", num_cores=NC, num_subcores=NS)


def _dense_body(emb_ref, ws_ref, bs_ref, wab_ref, bab_ref,
                item_ref, av_ref, bv_ref):
    y = jnp.dot(emb_ref[...], ws_ref[...],
                preferred_element_type=jnp.float32) + bs_ref[...]
    item_ref[...] = y
    aux = jnp.dot(y, wab_ref[...],
                  preferred_element_type=jnp.float32) + bab_ref[...]
    av_ref[...] = aux[:, 0:1]
    bv_ref[...] = aux[:, 1:2]


def _dense(emb_p, W_scale, b_scale, Wab, bab):
    return pl.pallas_call(
        _dense_body,
        grid=(N_PAD // RB,),
        in_specs=[
            pl.BlockSpec((RB, D), lambda i: (i, 0)),
            pl.BlockSpec((D, D), lambda i: (0, 0)),
            pl.BlockSpec((1, D), lambda i: (0, 0)),
            pl.BlockSpec((D, D), lambda i: (0, 0)),
            pl.BlockSpec((1, D), lambda i: (0, 0)),
        ],
        out_specs=[
            pl.BlockSpec((RB, D), lambda i: (i, 0)),
            pl.BlockSpec((RB, 1), lambda i: (i, 0)),
            pl.BlockSpec((RB, 1), lambda i: (i, 0)),
        ],
        out_shape=[
            jax.ShapeDtypeStruct((N_PAD, D), jnp.float32),
            jax.ShapeDtypeStruct((N_PAD, 1), jnp.float32),
            jax.ShapeDtypeStruct((N_PAD, 1), jnp.float32),
        ],
    )(emb_p, W_scale, b_scale, Wab, bab)


def _finish_body(num_ref, den_ref, o_ref):
    n = num_ref[0] + num_ref[1]
    d = den_ref[0] + den_ref[1]
    rd = 1.0 / jnp.maximum(d, 1e-37)
    o_ref[...] = jax.nn.sigmoid(n * rd)


def _finish(num, den2):
    return pl.pallas_call(
        _finish_body,
        grid=(N_PAD // RB,),
        in_specs=[
            pl.BlockSpec((NC, RB, D), lambda i: (0, i, 0)),
            pl.BlockSpec((NC, RB, 1), lambda i: (0, i, 0)),
        ],
        out_specs=pl.BlockSpec((RB, D), lambda i: (i, 0)),
        out_shape=jax.ShapeDtypeStruct((N_RAW, D), jnp.float32),
    )(num, den2)


def _edge_body(src_hbm, dst_hbm, av_hbm, bv_hbm, item_hbm,
               num_out, den_out,
               av_v, bv_v, src_c, dst_c, scat_c, s_v, rows_v,
               acc_sh, den_sh, sem_ix, sem_g, sem_d, sem_s):
    c = lax.axis_index("c")
    s = lax.axis_index("s")
    w = c * NS + s
    base = w * EPW
    zf = jnp.zeros((L,), jnp.float32)

    def idx_start(i, p):
        off = base + i * CH
        pltpu.async_copy(src_hbm.at[pl.ds(off, CH)], src_c[p], sem_ix[p])
        pltpu.async_copy(dst_hbm.at[pl.ds(off, CH)], dst_c[p], sem_ix[p])

    def idx_wait(p):
        pltpu.make_async_copy(src_hbm.at[pl.ds(0, CH)], src_c[p],
                              sem_ix[p]).wait()
        pltpu.make_async_copy(dst_hbm.at[pl.ds(0, CH)], dst_c[p],
                              sem_ix[p]).wait()

    def gather_start(p):
        pltpu.async_copy(item_hbm.at[dst_c[p]], rows_v[p], sem_g[p])

    def gather_wait(p):
        pltpu.make_async_copy(item_hbm.at[dst_c[p]], rows_v[p],
                              sem_g[p]).wait()

    def den_start(p):
        pltpu.async_copy(s_v[p], den_sh.at[scat_c[p]], sem_d[p], add=True)

    def den_wait(p):
        pltpu.make_async_copy(s_v[p], den_sh.at[scat_c[p]], sem_d[p]).wait()

    def scat_start(p):
        pltpu.async_copy(rows_v[p], acc_sh.at[scat_c[p]], sem_s[p], add=True)

    def scat_wait(p):
        pltpu.make_async_copy(rows_v[p], acc_sh.at[scat_c[p]],
                              sem_s[p]).wait()

    def compute_s(i, p):
        limit = EPW - i * CH
        for k in range(G):
            si = src_c[p][pl.ds(k * L, L)]
            di = dst_c[p][pl.ds(k * L, L)]
            scat_c[p][pl.ds(k * L, L)] = si
            att = plsc.load_gather(av_v, [si]) + plsc.load_gather(bv_v, [di])
            sc = jnp.exp(jnp.maximum(att, 0.2 * att) - 1.0)
            mask = (lax.iota(jnp.int32, L) + k * L) < limit
            s_v[p][pl.ds(k * L, L)] = jnp.where(mask, sc, 0.0)

    def scale_rows(p):
        for g in range(G):
            sv = s_v[p][pl.ds(g * L, L)]
            for jj in range(L):
                j = g * L + jj
                ssc = sv[jj]
                for k in range(D // L):
                    rows_v[p][j, pl.ds(k * L, L)] = (
                        rows_v[p][j, pl.ds(k * L, L)] * ssc)

    def zrow(j, _):
        for k in range(D // L):
            rows_v[0][j, pl.ds(k * L, L)] = zf
        return 0
    lax.fori_loop(0, CH, zrow, 0, unroll=False)
    for k in range(G):
        s_v[0][pl.ds(k * L, L)] = zf
    for k in range(6):
        pltpu.sync_copy(rows_v[0], acc_sh.at[pl.ds(s * RPT + k * CH, CH)])
        pltpu.sync_copy(s_v[0], den_sh.at[pl.ds(s * RPT + k * CH, CH)])
    pltpu.sync_copy(rows_v[0].at[pl.ds(0, RPT - 6 * CH)],
                    acc_sh.at[pl.ds(s * RPT + 6 * CH, RPT - 6 * CH)])
    pltpu.sync_copy(s_v[0].at[pl.ds(0, RPT - 6 * CH)],
                    den_sh.at[pl.ds(s * RPT + 6 * CH, RPT - 6 * CH)])

    pltpu.sync_copy(av_hbm, av_v)
    pltpu.sync_copy(bv_hbm, bv_v)
    idx_start(0, 0)
    idx_start(1, 1)
    idx_wait(0)
    gather_start(0)

    plsc.subcore_barrier()

    def pair(t, _):
        i = 2 * t
        idx_wait(1)

        @pl.when(t >= 1)
        def _():
            scat_wait(1)
        gather_start(1)

        @pl.when(t >= 1)
        def _():
            den_wait(0)
        compute_s(i, 0)
        den_start(0)
        gather_wait(0)
        scale_rows(0)
        scat_start(0)
        idx_start(i + 2, 0)

        idx_wait(0)
        scat_wait(0)
        gather_start(0)

        @pl.when(t >= 1)
        def _():
            den_wait(1)
        compute_s(i + 1, 1)
        den_start(1)
        gather_wait(1)
        scale_rows(1)
        scat_start(1)

        @pl.when(t <= NPAIR - 2)
        def _():
            idx_start(i + 3, 1)
        return 0
    lax.fori_loop(0, NPAIR, pair, 0, unroll=False)

    scat_wait(1)
    den_wait(0)
    compute_s(NCHUNK - 1, 0)
    den_start(0)
    gather_wait(0)
    scale_rows(0)
    scat_start(0)
    den_wait(1)
    den_wait(0)
    scat_wait(0)

    plsc.subcore_barrier()

    pltpu.sync_copy(acc_sh.at[pl.ds(s * RPT, RPT)],
                    num_out.at[c, pl.ds(s * RPT, RPT)])
    pltpu.sync_copy(den_sh.at[pl.ds(s * RPT, RPT)],
                    den_out.at[pl.ds(c * N_PAD + s * RPT, RPT)])


_edge_kernel = functools.partial(
    pl.kernel,
    compiler_params=pltpu.CompilerParams(needs_layout_passes=False),
    out_type=(
        jax.ShapeDtypeStruct((NC, N_PAD, D), jnp.float32),
        jax.ShapeDtypeStruct((NC * N_PAD,), jnp.float32),
    ),
    mesh=_MESH,
    scratch_types=[
        pltpu.VMEM((N_PAD,), jnp.float32),
        pltpu.VMEM((N_PAD,), jnp.float32),
        [pltpu.VMEM((CH,), jnp.int32)] * 2,
        [pltpu.VMEM((CH,), jnp.int32)] * 2,
        [pltpu.VMEM((CH,), jnp.int32)] * 2,
        [pltpu.VMEM((CH,), jnp.float32)] * 2,
        [pltpu.VMEM((CH, D), jnp.float32)] * 2,
        pltpu.VMEM_SHARED((N_PAD, D), jnp.float32),
        pltpu.VMEM_SHARED((N_PAD,), jnp.float32),
        [pltpu.SemaphoreType.DMA] * 2,
        [pltpu.SemaphoreType.DMA] * 2,
        [pltpu.SemaphoreType.DMA] * 2,
        [pltpu.SemaphoreType.DMA] * 2,
    ],
)(_edge_body)


def kernel(edge, emb_mat, W_scale, b_scale, W_att, b_att):
    src = edge[:, 0].astype(jnp.int32)
    dst = edge[:, 1].astype(jnp.int32)
    src = jnp.pad(src, (0, E_PAD - E))
    dst = jnp.pad(dst, (0, E_PAD - E))

    emb_p = jnp.pad(emb_mat, ((0, N_PAD - N_RAW), (0, 0)))
    Wab = jnp.zeros((D, D), jnp.float32)
    Wab = Wab.at[:, 0].set(W_att[:D, 0]).at[:, 1].set(W_att[D:, 0])
    bab = jnp.zeros((1, D), jnp.float32).at[0, 0].set(b_att[0])

    item, av, bv = _dense(emb_p, W_scale, b_scale.reshape(1, D), Wab, bab)

    num, den = _edge_kernel(src, dst, av.reshape(N_PAD), bv.reshape(N_PAD),
                            item)
    return _finish(num, den.reshape(NC, N_PAD, 1))
